# baseline (speedup 1.0000x reference)
import jax
import jax.numpy as jnp
from jax.experimental import pallas as pl

B, X, C = 1024, 2048, 16
SEG = 4
P, H = 64, 128
ROW = X * C
SEGN = ROW // SEG
BB = 16


def _router_body(u_ref, ds_ref, pde_ref, W1_ref, b1_ref, W2_ref, b2_ref,
                 emb_ref, out_ref):
    x = u_ref[...]
    seg_sum, seg_sq = [], []
    for s in range(SEG):
        xs = x[:, s * SEGN:(s + 1) * SEGN]
        seg_sum.append(jnp.sum(xs, axis=1, keepdims=True))
        seg_sq.append(jnp.sum(xs * xs, axis=1, keepdims=True))
    tot_sum = (seg_sum[0] + seg_sum[1]) + (seg_sum[2] + seg_sum[3])
    tot_sq = (seg_sq[0] + seg_sq[1]) + (seg_sq[2] + seg_sq[3])
    mn = jnp.min(x, axis=1, keepdims=True)
    mx = jnp.max(x, axis=1, keepdims=True)
    d = x[:, C:] - x[:, :-C]
    absd = jnp.sum(jnp.abs(d), axis=1, keepdims=True)

    nf = float(ROW)
    mean = tot_sum * (1.0 / nf)
    var = (tot_sq - tot_sum * tot_sum * (1.0 / nf)) * (1.0 / (nf - 1.0))
    std = jnp.sqrt(jnp.maximum(var, 0.0))
    gmean = absd * (1.0 / float(ROW - C))
    sf = float(SEGN)
    smeans = [s_ * (1.0 / sf) for s_ in seg_sum]
    sstds = [jnp.sqrt(jnp.maximum((q - s_ * s_ * (1.0 / sf)) * (1.0 / (sf - 1.0)), 0.0))
             for q, s_ in zip(seg_sq, seg_sum)]

    pde = pde_ref[...]
    ds = ds_ref[...]
    onehot = (ds == jax.lax.broadcasted_iota(jnp.int32, (BB, 4), 1)
              ).astype(jnp.float32)
    dse = jnp.dot(onehot, emb_ref[...], preferred_element_type=jnp.float32)

    feat = jnp.concatenate(
        [mean, std, mn, mx, gmean] + smeans + sstds
        + [pde, dse, jnp.zeros((BB, 3), jnp.float32)], axis=1)
    h = jnp.dot(feat, W1_ref[...], preferred_element_type=jnp.float32) + b1_ref[...]
    h = jnp.maximum(h, 0.0)
    logits = jnp.dot(h, W2_ref[...], preferred_element_type=jnp.float32) + b2_ref[...]

    idx = jax.lax.broadcasted_iota(jnp.int32, (BB, P), 1)
    m1 = jnp.max(logits, axis=1, keepdims=True)
    i1 = jnp.min(jnp.where(logits == m1, idx, P), axis=1, keepdims=True)
    lmask = jnp.where(idx == i1, -jnp.inf, logits)
    m2 = jnp.max(lmask, axis=1, keepdims=True)
    i2 = jnp.min(jnp.where(lmask == m2, idx, P), axis=1, keepdims=True)
    has = (i1 == 0) | (i2 == 0)
    i2f = jnp.where(has, i2, 0)
    lt2 = jnp.where(has, m2, logits[:, 0:1])
    a = jnp.maximum(m1, lt2)
    e1 = jnp.exp(m1 - a)
    e2 = jnp.exp(lt2 - a)
    inv = 1.0 / (e1 + e2)
    out_ref[...] = (e1 * inv) * (idx == i1).astype(jnp.float32) \
        + (e2 * inv) * (idx == i2f).astype(jnp.float32)


def kernel(u_state, pde_params, dataset_id, W1, b1, W2, b2, emb):
    u2 = u_state.reshape(B, ROW)
    ds2 = dataset_id.astype(jnp.int32).reshape(B, 1)
    W1p = jnp.concatenate([W1, jnp.zeros((3, H), W1.dtype)], axis=0)
    b1r = b1.reshape(1, H)
    b2r = b2.reshape(1, P)
    return pl.pallas_call(
        _router_body,
        grid=(B // BB,),
        in_specs=[
            pl.BlockSpec((BB, ROW), lambda i: (i, 0)),
            pl.BlockSpec((BB, 1), lambda i: (i, 0)),
            pl.BlockSpec((BB, 8), lambda i: (i, 0)),
            pl.BlockSpec((32, H), lambda i: (0, 0)),
            pl.BlockSpec((1, H), lambda i: (0, 0)),
            pl.BlockSpec((H, P), lambda i: (0, 0)),
            pl.BlockSpec((1, P), lambda i: (0, 0)),
            pl.BlockSpec((4, 8), lambda i: (0, 0)),
        ],
        out_specs=pl.BlockSpec((BB, P), lambda i: (i, 0)),
        out_shape=jax.ShapeDtypeStruct((B, P), jnp.float32),
    )(u2, ds2, pde_params, W1p, b1r, W2, b2r, emb)


# speedup vs baseline: 2.2659x; 2.2659x over previous
import jax
import jax.numpy as jnp
from jax.experimental import pallas as pl

B, X, C = 1024, 2048, 16
SEG = 4
P, H = 64, 128
XS = X // SEG
BB = 16


def _router_body(u_ref, ds_ref, pde_ref, W1_ref, b1_ref, W2_ref, b2_ref,
                 emb_ref, out_ref):
    x = u_ref[...]
    TL = 128
    seg_sum, seg_sq, seg_mn, seg_mx = [], [], [], []
    for s in range(SEG):
        p = [x[:, :, s * XS + k * TL:s * XS + (k + 1) * TL]
             for k in range(XS // TL)]
        a = (p[0] + p[1]) + (p[2] + p[3])
        q = (p[0] * p[0] + p[1] * p[1]) + (p[2] * p[2] + p[3] * p[3])
        seg_mn.append(jnp.minimum(jnp.minimum(p[0], p[1]),
                                  jnp.minimum(p[2], p[3])))
        seg_mx.append(jnp.maximum(jnp.maximum(p[0], p[1]),
                                  jnp.maximum(p[2], p[3])))
        seg_sum.append(jnp.sum(a, axis=(1, 2), keepdims=True).reshape(BB, 1))
        seg_sq.append(jnp.sum(q, axis=(1, 2), keepdims=True).reshape(BB, 1))
    tot_sum = (seg_sum[0] + seg_sum[1]) + (seg_sum[2] + seg_sum[3])
    tot_sq = (seg_sq[0] + seg_sq[1]) + (seg_sq[2] + seg_sq[3])
    mn_f = jnp.minimum(jnp.minimum(seg_mn[0], seg_mn[1]),
                       jnp.minimum(seg_mn[2], seg_mn[3]))
    mx_f = jnp.maximum(jnp.maximum(seg_mx[0], seg_mx[1]),
                       jnp.maximum(seg_mx[2], seg_mx[3]))
    mn = jnp.min(mn_f, axis=(1, 2), keepdims=True).reshape(BB, 1)
    mx = jnp.max(mx_f, axis=(1, 2), keepdims=True).reshape(BB, 1)
    absd = jnp.sum(jnp.abs(x[:, :, 1:] - x[:, :, :-1]),
                   axis=(1, 2), keepdims=True).reshape(BB, 1)

    nf = float(X * C)
    mean = tot_sum * (1.0 / nf)
    var = (tot_sq - tot_sum * tot_sum * (1.0 / nf)) * (1.0 / (nf - 1.0))
    std = jnp.sqrt(jnp.maximum(var, 0.0))
    gmean = absd * (1.0 / float((X - 1) * C))
    sf = float(XS * C)
    smeans = [s_ * (1.0 / sf) for s_ in seg_sum]
    sstds = [jnp.sqrt(jnp.maximum((q - s_ * s_ * (1.0 / sf)) * (1.0 / (sf - 1.0)), 0.0))
             for q, s_ in zip(seg_sq, seg_sum)]

    pde = pde_ref[...]
    ds = ds_ref[...]
    onehot = (ds == jax.lax.broadcasted_iota(jnp.int32, (BB, 4), 1)
              ).astype(jnp.float32)
    dse = jnp.dot(onehot, emb_ref[...], preferred_element_type=jnp.float32)

    feat = jnp.concatenate(
        [mean, std, mn, mx, gmean] + smeans + sstds
        + [pde, dse, jnp.zeros((BB, 3), jnp.float32)], axis=1)
    h = jnp.dot(feat, W1_ref[...], preferred_element_type=jnp.float32) + b1_ref[...]
    h = jnp.maximum(h, 0.0)
    logits = jnp.dot(h, W2_ref[...], preferred_element_type=jnp.float32) + b2_ref[...]

    idx = jax.lax.broadcasted_iota(jnp.int32, (BB, P), 1)
    m1 = jnp.max(logits, axis=1, keepdims=True)
    i1 = jnp.min(jnp.where(logits == m1, idx, P), axis=1, keepdims=True)
    lmask = jnp.where(idx == i1, -jnp.inf, logits)
    m2 = jnp.max(lmask, axis=1, keepdims=True)
    i2 = jnp.min(jnp.where(lmask == m2, idx, P), axis=1, keepdims=True)
    has = (i1 == 0) | (i2 == 0)
    i2f = jnp.where(has, i2, 0)
    lt2 = jnp.where(has, m2, logits[:, 0:1])
    a = jnp.maximum(m1, lt2)
    e1 = jnp.exp(m1 - a)
    e2 = jnp.exp(lt2 - a)
    inv = 1.0 / (e1 + e2)
    out_ref[...] = (e1 * inv) * (idx == i1).astype(jnp.float32) \
        + (e2 * inv) * (idx == i2f).astype(jnp.float32)


def kernel(u_state, pde_params, dataset_id, W1, b1, W2, b2, emb):
    ut = jnp.transpose(u_state, (0, 2, 1))
    ds2 = dataset_id.astype(jnp.int32).reshape(B, 1)
    W1p = jnp.concatenate([W1, jnp.zeros((3, H), W1.dtype)], axis=0)
    b1r = b1.reshape(1, H)
    b2r = b2.reshape(1, P)
    return pl.pallas_call(
        _router_body,
        grid=(B // BB,),
        in_specs=[
            pl.BlockSpec((BB, C, X), lambda i: (i, 0, 0)),
            pl.BlockSpec((BB, 1), lambda i: (i, 0)),
            pl.BlockSpec((BB, 8), lambda i: (i, 0)),
            pl.BlockSpec((32, H), lambda i: (0, 0)),
            pl.BlockSpec((1, H), lambda i: (0, 0)),
            pl.BlockSpec((H, P), lambda i: (0, 0)),
            pl.BlockSpec((1, P), lambda i: (0, 0)),
            pl.BlockSpec((4, 8), lambda i: (0, 0)),
        ],
        out_specs=pl.BlockSpec((BB, P), lambda i: (i, 0)),
        out_shape=jax.ShapeDtypeStruct((B, P), jnp.float32),
    )(ut, ds2, pde_params, W1p, b1r, W2, b2r, emb)


# speedup vs baseline: 3.0744x; 1.3568x over previous
import jax
import jax.numpy as jnp
from jax.experimental import pallas as pl
from jax.experimental.pallas import tpu as pltpu

B, X, C = 1024, 2048, 16
SEG = 4
P, H = 64, 128
XS = X // SEG
BB = 32


def _router_body(u_ref, ds_ref, pde_ref, W1_ref, b1_ref, W2_ref, b2_ref,
                 emb_ref, out_ref):
    TL = 128
    NT = X // TL
    lane = jax.lax.broadcasted_iota(jnp.int32, (1, 1, TL), 2)
    last = lane == TL - 1
    p = [u_ref[:, :, k * TL:(k + 1) * TL] for k in range(NT)]
    r = [pltpu.roll(pk, TL - 1, 2) for pk in p]
    dseg = []
    for k in range(NT):
        if k + 1 < NT:
            nb = jnp.where(last, r[k + 1], r[k])
            dseg.append(jnp.abs(nb - p[k]))
        else:
            dseg.append(jnp.abs(r[k] - p[k]) * (1.0 - last.astype(jnp.float32)))
    seg_sum, seg_sq, seg_mn, seg_mx = [], [], [], []
    for s in range(SEG):
        ps = p[4 * s:4 * s + 4]
        a = (ps[0] + ps[1]) + (ps[2] + ps[3])
        q = (ps[0] * ps[0] + ps[1] * ps[1]) + (ps[2] * ps[2] + ps[3] * ps[3])
        seg_mn.append(jnp.minimum(jnp.minimum(ps[0], ps[1]),
                                  jnp.minimum(ps[2], ps[3])))
        seg_mx.append(jnp.maximum(jnp.maximum(ps[0], ps[1]),
                                  jnp.maximum(ps[2], ps[3])))
        seg_sum.append(jnp.sum(a, axis=(1, 2), keepdims=True).reshape(BB, 1))
        seg_sq.append(jnp.sum(q, axis=(1, 2), keepdims=True).reshape(BB, 1))
    tot_sum = (seg_sum[0] + seg_sum[1]) + (seg_sum[2] + seg_sum[3])
    tot_sq = (seg_sq[0] + seg_sq[1]) + (seg_sq[2] + seg_sq[3])
    mn_f = jnp.minimum(jnp.minimum(seg_mn[0], seg_mn[1]),
                       jnp.minimum(seg_mn[2], seg_mn[3]))
    mx_f = jnp.maximum(jnp.maximum(seg_mx[0], seg_mx[1]),
                       jnp.maximum(seg_mx[2], seg_mx[3]))
    mn = jnp.min(mn_f, axis=(1, 2), keepdims=True).reshape(BB, 1)
    mx = jnp.max(mx_f, axis=(1, 2), keepdims=True).reshape(BB, 1)
    ad = [(dseg[4 * k] + dseg[4 * k + 1]) + (dseg[4 * k + 2] + dseg[4 * k + 3])
          for k in range(4)]
    ad_f = (ad[0] + ad[1]) + (ad[2] + ad[3])
    absd = jnp.sum(ad_f, axis=(1, 2), keepdims=True).reshape(BB, 1)

    nf = float(X * C)
    mean = tot_sum * (1.0 / nf)
    var = (tot_sq - tot_sum * tot_sum * (1.0 / nf)) * (1.0 / (nf - 1.0))
    std = jnp.sqrt(jnp.maximum(var, 0.0))
    gmean = absd * (1.0 / float((X - 1) * C))
    sf = float(XS * C)
    smeans = [s_ * (1.0 / sf) for s_ in seg_sum]
    sstds = [jnp.sqrt(jnp.maximum((q - s_ * s_ * (1.0 / sf)) * (1.0 / (sf - 1.0)), 0.0))
             for q, s_ in zip(seg_sq, seg_sum)]

    pde = pde_ref[...]
    ds = ds_ref[...]
    onehot = (ds == jax.lax.broadcasted_iota(jnp.int32, (BB, 4), 1)
              ).astype(jnp.float32)
    dse = jnp.dot(onehot, emb_ref[...], preferred_element_type=jnp.float32)

    feat = jnp.concatenate(
        [mean, std, mn, mx, gmean] + smeans + sstds
        + [pde, dse, jnp.zeros((BB, 3), jnp.float32)], axis=1)
    h = jnp.dot(feat, W1_ref[...], preferred_element_type=jnp.float32) + b1_ref[...]
    h = jnp.maximum(h, 0.0)
    logits = jnp.dot(h, W2_ref[...], preferred_element_type=jnp.float32) + b2_ref[...]

    idx = jax.lax.broadcasted_iota(jnp.int32, (BB, P), 1)
    m1 = jnp.max(logits, axis=1, keepdims=True)
    i1 = jnp.min(jnp.where(logits == m1, idx, P), axis=1, keepdims=True)
    lmask = jnp.where(idx == i1, -jnp.inf, logits)
    m2 = jnp.max(lmask, axis=1, keepdims=True)
    i2 = jnp.min(jnp.where(lmask == m2, idx, P), axis=1, keepdims=True)
    has = (i1 == 0) | (i2 == 0)
    i2f = jnp.where(has, i2, 0)
    lt2 = jnp.where(has, m2, logits[:, 0:1])
    a = jnp.maximum(m1, lt2)
    e1 = jnp.exp(m1 - a)
    e2 = jnp.exp(lt2 - a)
    inv = 1.0 / (e1 + e2)
    out_ref[...] = (e1 * inv) * (idx == i1).astype(jnp.float32) \
        + (e2 * inv) * (idx == i2f).astype(jnp.float32)


def kernel(u_state, pde_params, dataset_id, W1, b1, W2, b2, emb):
    ut = jnp.transpose(u_state, (0, 2, 1))
    ds2 = dataset_id.astype(jnp.int32).reshape(B, 1)
    W1p = jnp.concatenate([W1, jnp.zeros((3, H), W1.dtype)], axis=0)
    b1r = b1.reshape(1, H)
    b2r = b2.reshape(1, P)
    return pl.pallas_call(
        _router_body,
        grid=(B // BB,),
        in_specs=[
            pl.BlockSpec((BB, C, X), lambda i: (i, 0, 0)),
            pl.BlockSpec((BB, 1), lambda i: (i, 0)),
            pl.BlockSpec((BB, 8), lambda i: (i, 0)),
            pl.BlockSpec((32, H), lambda i: (0, 0)),
            pl.BlockSpec((1, H), lambda i: (0, 0)),
            pl.BlockSpec((H, P), lambda i: (0, 0)),
            pl.BlockSpec((1, P), lambda i: (0, 0)),
            pl.BlockSpec((4, 8), lambda i: (0, 0)),
        ],
        out_specs=pl.BlockSpec((BB, P), lambda i: (i, 0)),
        out_shape=jax.ShapeDtypeStruct((B, P), jnp.float32),
    )(ut, ds2, pde_params, W1p, b1r, W2, b2r, emb)
